# baseline (speedup 1.0000x reference)
import functools

import jax
import jax.numpy as jnp
from jax import lax
from jax.experimental import pallas as pl
from jax.experimental.pallas import tpu as pltpu
from jax.experimental.pallas import tpu_sc as plsc

_EPS = 1e-6
_T = 256
_NCOMBO = 6


def _router_body(x_ref, wr_ref, br_ref, pos_ref, lo_ref, hi_ref,
                 cmb_s, rnk_s):
    i = pl.program_id(0)
    NT = pl.num_programs(0) - 1
    E, D = wr_ref.shape
    T = x_ref.shape[0]
    GT = lo_ref.shape[1]

    @pl.when(i < NT)
    def _route_tile():
        xb = x_ref[:].astype(jnp.bfloat16)
        logt = lax.dot_general(
            wr_ref[:], xb, (((1,), (1,)), ((), ())),
            preferred_element_type=jnp.float32) + br_ref[:]
        iota_e = lax.broadcasted_iota(jnp.int32, (E, T), 0)
        v1 = jnp.max(logt, axis=0, keepdims=True)
        i1 = jnp.min(jnp.where(logt == v1, iota_e, E), axis=0, keepdims=True)
        masked = jnp.where(iota_e == i1, -jnp.inf, logt)
        v2 = jnp.max(masked, axis=0, keepdims=True)
        i2 = jnp.min(jnp.where(masked == v2, iota_e, E), axis=0, keepdims=True)
        lo = jnp.minimum(i1, i2)
        hi = jnp.maximum(i1, i2)
        c = lo * (7 - lo) // 2 + hi - lo - 1
        iota_c = lax.broadcasted_iota(jnp.int32, (_NCOMBO, T), 0)
        oh = (iota_c == c).astype(jnp.int32)
        inc = oh
        for sh in (1, 2, 4, 8, 16, 32, 64, 128):
            if sh < T:
                inc = inc + jnp.concatenate(
                    [jnp.zeros((_NCOMBO, sh), jnp.int32), inc[:, :-sh]], axis=1)
        excl = inc - oh
        rank = jnp.sum(jnp.where(iota_c == c, excl, 0), axis=0, keepdims=True)
        cmb_s[pl.ds(i, 1), :] = c
        rnk_s[pl.ds(i, 1), :] = rank

    @pl.when(i == NT)
    def _finalize():
        cmb = cmb_s[:]
        rnk = rnk_s[:]
        cols = [jnp.sum((cmb == j).astype(jnp.int32), axis=1, keepdims=True)
                for j in range(_NCOMBO)]
        counts = jnp.concatenate(cols, axis=1)
        inc = counts
        sh = 1
        while sh < NT:
            inc = inc + jnp.concatenate(
                [jnp.zeros((sh, _NCOMBO), jnp.int32), inc[:-sh]], axis=0)
            sh *= 2
        excl_tiles = inc - counts
        totals = inc[NT - 1:NT, :]
        ps = ((totals + (T - 1)) // T) * T
        incp = ps
        for sh in (1, 2, 4):
            incp = incp + jnp.concatenate(
                [jnp.zeros((1, sh), jnp.int32), incp[:, :-sh]], axis=1)
        po = incp - ps
        base = excl_tiles + po
        pos = rnk
        for j in range(_NCOMBO):
            pos = pos + jnp.where(cmb == j, base[:, j:j + 1], 0)
        pos_ref[:] = pos

        end_tiles = (po + ps) // T
        t_iota = lax.broadcasted_iota(jnp.int32, (1, GT), 1)
        cot = jnp.zeros((1, GT), jnp.int32)
        for j in range(_NCOMBO):
            cot = cot + (t_iota >= end_tiles[:, j:j + 1]).astype(jnp.int32)
        cot = jnp.minimum(cot, _NCOMBO - 1)
        lo_t = jnp.where(cot < 3, 0, jnp.where(cot < 5, 1, 2))
        blo = (lo_t * (7 - lo_t)) // 2
        hi_t = cot - blo + lo_t + 1
        lo_ref[:] = lo_t
        hi_ref[:] = hi_t


def _expert_body(lo_sref, hi_sref, x_ref, wr_ref, br_ref, welo_ref, wehi_ref,
                 be_ref, wo_ref, bo_ref, nw_ref, o_ref):
    i = pl.program_id(0)
    lo = lo_sref[i]
    hi = hi_sref[i]
    T, D = x_ref.shape
    E = wr_ref.shape[0]

    xf = x_ref[:]
    xb = xf.astype(jnp.bfloat16)
    logits = lax.dot_general(
        xb, wr_ref[:], (((1,), (1,)), ((), ())),
        preferred_element_type=jnp.float32) + br_ref[:]
    idx = lax.broadcasted_iota(jnp.int32, (T, E), 1)
    v1 = jnp.max(logits, axis=1, keepdims=True)
    i1 = jnp.min(jnp.where(logits == v1, idx, E), axis=1, keepdims=True)
    masked = jnp.where(idx == i1, -jnp.inf, logits)
    v2 = jnp.max(masked, axis=1, keepdims=True)
    i2 = jnp.min(jnp.where(masked == v2, idx, E), axis=1, keepdims=True)
    s = jnp.exp(v2 - v1)
    w1 = 1.0 / (1.0 + s)
    w2 = s * w1
    gates = jnp.where(idx == i1, w1, 0.0) + jnp.where(idx == i2, w2, 0.0)
    wlo = jnp.sum(jnp.where(idx == lo, gates, 0.0), axis=1, keepdims=True)
    whi = jnp.sum(jnp.where(idx == hi, gates, 0.0), axis=1, keepdims=True)

    hlo = lax.dot_general(
        xb, welo_ref[0], (((1,), (1,)), ((), ())),
        preferred_element_type=jnp.float32) + be_ref[pl.ds(lo, 1), :]
    hhi = lax.dot_general(
        xb, wehi_ref[0], (((1,), (1,)), ((), ())),
        preferred_element_type=jnp.float32) + be_ref[pl.ds(hi, 1), :]
    h = wlo * (hlo * jax.nn.sigmoid(hlo)) + whi * (hhi * jax.nn.sigmoid(hhi))

    ob = lax.dot_general(
        h.astype(jnp.bfloat16), wo_ref[:], (((1,), (1,)), ((), ())),
        preferred_element_type=jnp.float32) + bo_ref[:]
    y = xf + ob
    r = lax.rsqrt(jnp.mean(y * y, axis=1, keepdims=True) + _EPS)
    o_ref[:] = (nw_ref[:] * y) * r


def _make_sc_scatter(n_rows, cap, d, chunk):
    mesh = plsc.VectorSubcoreMesh(core_axis_name="c", subcore_axis_name="s")
    nw = 32
    per_w = n_rows // nw

    @functools.partial(
        pl.kernel,
        out_type=jax.ShapeDtypeStruct((cap, d), jnp.float32),
        mesh=mesh,
        scratch_types=[
            pltpu.VMEM((chunk,), jnp.int32),
            pltpu.VMEM((chunk, d), jnp.float32),
            pltpu.SemaphoreType.DMA,
        ],
    )
    def sc_scatter(x_hbm, pos_hbm, xs_hbm, idx_v, rows_v, sem):
        wid = lax.axis_index("s") * 2 + lax.axis_index("c")
        base = wid * per_w
        for j in range(per_w // chunk):
            off = base + j * chunk
            pltpu.sync_copy(pos_hbm.at[pl.ds(off, chunk)], idx_v)
            pltpu.sync_copy(x_hbm.at[pl.ds(off, chunk)], rows_v)
            pltpu.async_copy(rows_v, xs_hbm.at[idx_v], sem).wait()

    return sc_scatter


def _make_sc_gather(n_rows, cap, d, chunk):
    mesh = plsc.VectorSubcoreMesh(core_axis_name="c", subcore_axis_name="s")
    nw = 32
    per_w = n_rows // nw

    @functools.partial(
        pl.kernel,
        out_type=jax.ShapeDtypeStruct((n_rows, d), jnp.float32),
        mesh=mesh,
        scratch_types=[
            pltpu.VMEM((chunk,), jnp.int32),
            pltpu.VMEM((chunk, d), jnp.float32),
            pltpu.SemaphoreType.DMA,
        ],
    )
    def sc_gather(ys_hbm, pos_hbm, out_hbm, idx_v, rows_v, sem):
        wid = lax.axis_index("s") * 2 + lax.axis_index("c")
        base = wid * per_w
        for j in range(per_w // chunk):
            off = base + j * chunk
            pltpu.sync_copy(pos_hbm.at[pl.ds(off, chunk)], idx_v)
            pltpu.async_copy(ys_hbm.at[idx_v], rows_v, sem).wait()
            pltpu.sync_copy(rows_v, out_hbm.at[pl.ds(off, chunk)])

    return sc_gather


@jax.jit
def kernel(x, Wr, br, We, be, Wo, bo, norm_w):
    B, S, D = x.shape
    E = Wr.shape[0]
    N = B * S
    NT = N // _T
    CAP = N + _NCOMBO * _T
    GT = CAP // _T

    xf = x.reshape(N, D)
    Wr16 = Wr.astype(jnp.bfloat16)
    We16 = We.astype(jnp.bfloat16)
    Wo16 = Wo.astype(jnp.bfloat16)
    brT = br.reshape(E, 1)
    br2 = br.reshape(1, E)
    bo2 = bo.reshape(1, D)
    nw2 = norm_w.reshape(1, D)

    pos_arr, lo_t, hi_t = pl.pallas_call(
        _router_body,
        grid=(NT + 1,),
        in_specs=[
            pl.BlockSpec((_T, D), lambda i: (jnp.minimum(i, NT - 1), 0)),
            pl.BlockSpec((E, D), lambda i: (0, 0)),
            pl.BlockSpec((E, 1), lambda i: (0, 0)),
        ],
        out_specs=[
            pl.BlockSpec((NT, _T), lambda i: (0, 0)),
            pl.BlockSpec((1, GT), lambda i: (0, 0)),
            pl.BlockSpec((1, GT), lambda i: (0, 0)),
        ],
        out_shape=[
            jax.ShapeDtypeStruct((NT, _T), jnp.int32),
            jax.ShapeDtypeStruct((1, GT), jnp.int32),
            jax.ShapeDtypeStruct((1, GT), jnp.int32),
        ],
        scratch_shapes=[
            pltpu.VMEM((NT, _T), jnp.int32),
            pltpu.VMEM((NT, _T), jnp.int32),
        ],
    )(xf, Wr16, brT)

    pos = pos_arr.reshape(N)
    xs = _make_sc_scatter(N, CAP, D, 32)(xf, pos)

    ys = pl.pallas_call(
        _expert_body,
        grid_spec=pltpu.PrefetchScalarGridSpec(
            num_scalar_prefetch=2,
            grid=(GT,),
            in_specs=[
                pl.BlockSpec((_T, D), lambda i, lo, hi: (i, 0)),
                pl.BlockSpec((E, D), lambda i, lo, hi: (0, 0)),
                pl.BlockSpec((1, E), lambda i, lo, hi: (0, 0)),
                pl.BlockSpec((1, D, D), lambda i, lo, hi: (lo[i], 0, 0)),
                pl.BlockSpec((1, D, D), lambda i, lo, hi: (hi[i], 0, 0)),
                pl.BlockSpec((E, D), lambda i, lo, hi: (0, 0)),
                pl.BlockSpec((D, D), lambda i, lo, hi: (0, 0)),
                pl.BlockSpec((1, D), lambda i, lo, hi: (0, 0)),
                pl.BlockSpec((1, D), lambda i, lo, hi: (0, 0)),
            ],
            out_specs=pl.BlockSpec((_T, D), lambda i, lo, hi: (i, 0)),
        ),
        out_shape=jax.ShapeDtypeStruct((CAP, D), jnp.float32),
    )(lo_t.reshape(GT), hi_t.reshape(GT),
      xs, Wr16, br2, We16, We16, be, Wo16, bo2, nw2)

    out = _make_sc_gather(N, CAP, D, 32)(ys, pos)
    return out.reshape(B, S, D)


# speedup vs baseline: 1.0100x; 1.0100x over previous
import functools

import jax
import jax.numpy as jnp
from jax import lax
from jax.experimental import pallas as pl
from jax.experimental.pallas import tpu as pltpu
from jax.experimental.pallas import tpu_sc as plsc

_EPS = 1e-6
_T = 256
_NCOMBO = 6
_NW = 32


def _router_body(x_ref, wr_ref, br_ref, pos_ref, lo_ref, hi_ref,
                 cmb_s, rnk_s):
    i = pl.program_id(0)
    NT = pl.num_programs(0) - 1
    E, D = wr_ref.shape
    T = x_ref.shape[0]
    GT = lo_ref.shape[1]

    @pl.when(i < NT)
    def _route_tile():
        xb = x_ref[:].astype(jnp.bfloat16)
        logt = lax.dot_general(
            wr_ref[:], xb, (((1,), (1,)), ((), ())),
            preferred_element_type=jnp.float32) + br_ref[:]
        iota_e = lax.broadcasted_iota(jnp.int32, (E, T), 0)
        v1 = jnp.max(logt, axis=0, keepdims=True)
        i1 = jnp.min(jnp.where(logt == v1, iota_e, E), axis=0, keepdims=True)
        masked = jnp.where(iota_e == i1, -jnp.inf, logt)
        v2 = jnp.max(masked, axis=0, keepdims=True)
        i2 = jnp.min(jnp.where(masked == v2, iota_e, E), axis=0, keepdims=True)
        lo = jnp.minimum(i1, i2)
        hi = jnp.maximum(i1, i2)
        c = lo * (7 - lo) // 2 + hi - lo - 1
        iota_c = lax.broadcasted_iota(jnp.int32, (_NCOMBO, T), 0)
        oh = (iota_c == c).astype(jnp.int32)
        inc = oh
        for sh in (1, 2, 4, 8, 16, 32, 64, 128):
            if sh < T:
                inc = inc + jnp.concatenate(
                    [jnp.zeros((_NCOMBO, sh), jnp.int32), inc[:, :-sh]], axis=1)
        excl = inc - oh
        rank = jnp.sum(jnp.where(iota_c == c, excl, 0), axis=0, keepdims=True)
        cmb_s[pl.ds(i, 1), :] = c
        rnk_s[pl.ds(i, 1), :] = rank

    @pl.when(i == NT)
    def _finalize():
        cmb = cmb_s[:]
        rnk = rnk_s[:]
        cols = [jnp.sum((cmb == j).astype(jnp.int32), axis=1, keepdims=True)
                for j in range(_NCOMBO)]
        counts = jnp.concatenate(cols, axis=1)
        inc = counts
        sh = 1
        while sh < NT:
            inc = inc + jnp.concatenate(
                [jnp.zeros((sh, _NCOMBO), jnp.int32), inc[:-sh]], axis=0)
            sh *= 2
        excl_tiles = inc - counts
        totals = inc[NT - 1:NT, :]
        ps = ((totals + (T - 1)) // T) * T
        incp = ps
        for sh in (1, 2, 4):
            incp = incp + jnp.concatenate(
                [jnp.zeros((1, sh), jnp.int32), incp[:, :-sh]], axis=1)
        po = incp - ps
        base = excl_tiles + po
        pos = rnk
        for j in range(_NCOMBO):
            pos = pos + jnp.where(cmb == j, base[:, j:j + 1], 0)
        pos_ref[:] = pos

        end_tiles = (po + ps) // T
        t_iota = lax.broadcasted_iota(jnp.int32, (1, GT), 1)
        cot = jnp.zeros((1, GT), jnp.int32)
        for j in range(_NCOMBO):
            cot = cot + (t_iota >= end_tiles[:, j:j + 1]).astype(jnp.int32)
        cot = jnp.minimum(cot, _NCOMBO - 1)
        lo_t = jnp.where(cot < 3, 0, jnp.where(cot < 5, 1, 2))
        blo = (lo_t * (7 - lo_t)) // 2
        hi_t = cot - blo + lo_t + 1
        lo_ref[:] = lo_t
        hi_ref[:] = hi_t


def _expert_body(lo_sref, hi_sref, x_ref, wr_ref, br_ref, welo_ref, wehi_ref,
                 be_ref, wo_ref, bo_ref, nw_ref, o_ref):
    i = pl.program_id(0)
    lo = lo_sref[i]
    hi = hi_sref[i]
    T, D = x_ref.shape
    E = wr_ref.shape[0]

    xf = x_ref[:]
    xb = xf.astype(jnp.bfloat16)
    logits = lax.dot_general(
        xb, wr_ref[:], (((1,), (1,)), ((), ())),
        preferred_element_type=jnp.float32) + br_ref[:]
    idx = lax.broadcasted_iota(jnp.int32, (T, E), 1)
    v1 = jnp.max(logits, axis=1, keepdims=True)
    i1 = jnp.min(jnp.where(logits == v1, idx, E), axis=1, keepdims=True)
    masked = jnp.where(idx == i1, -jnp.inf, logits)
    v2 = jnp.max(masked, axis=1, keepdims=True)
    i2 = jnp.min(jnp.where(masked == v2, idx, E), axis=1, keepdims=True)
    s = jnp.exp(v2 - v1)
    w1 = 1.0 / (1.0 + s)
    w2 = s * w1
    gates = jnp.where(idx == i1, w1, 0.0) + jnp.where(idx == i2, w2, 0.0)
    wlo = jnp.sum(jnp.where(idx == lo, gates, 0.0), axis=1, keepdims=True)
    whi = jnp.sum(jnp.where(idx == hi, gates, 0.0), axis=1, keepdims=True)

    hlo = lax.dot_general(
        xb, welo_ref[0], (((1,), (1,)), ((), ())),
        preferred_element_type=jnp.float32) + be_ref[pl.ds(lo, 1), :]
    hhi = lax.dot_general(
        xb, wehi_ref[0], (((1,), (1,)), ((), ())),
        preferred_element_type=jnp.float32) + be_ref[pl.ds(hi, 1), :]
    h = wlo * (hlo * jax.nn.sigmoid(hlo)) + whi * (hhi * jax.nn.sigmoid(hhi))

    ob = lax.dot_general(
        h.astype(jnp.bfloat16), wo_ref[:], (((1,), (1,)), ((), ())),
        preferred_element_type=jnp.float32) + bo_ref[:]
    y = xf + ob
    r = lax.rsqrt(jnp.mean(y * y, axis=1, keepdims=True) + _EPS)
    o_ref[:] = (nw_ref[:] * y) * r


def _make_sc_scatter(n_rows, cap, d, chunk):
    mesh = plsc.VectorSubcoreMesh(core_axis_name="c", subcore_axis_name="s")
    per_w = n_rows // _NW
    nchunk = per_w // chunk

    @functools.partial(
        pl.kernel,
        out_type=jax.ShapeDtypeStruct((cap, d), jnp.float32),
        mesh=mesh,
        scratch_types=[
            pltpu.VMEM((chunk,), jnp.int32),
            pltpu.VMEM((chunk,), jnp.int32),
            pltpu.VMEM((chunk, d), jnp.float32),
            pltpu.VMEM((chunk, d), jnp.float32),
            pltpu.SemaphoreType.DMA,
            pltpu.SemaphoreType.DMA,
        ],
    )
    def sc_scatter(x_hbm, pos_hbm, xs_hbm, idx0, idx1, rb0, rb1, sem0, sem1):
        wid = lax.axis_index("s") * 2 + lax.axis_index("c")
        base = wid * per_w
        idxs = (idx0, idx1)
        rbs = (rb0, rb1)
        sems = (sem0, sem1)
        pltpu.sync_copy(pos_hbm.at[pl.ds(base, chunk)], idx0)
        pltpu.sync_copy(x_hbm.at[pl.ds(base, chunk)], rb0)
        for j in range(nchunk):
            cur = j % 2
            cp = pltpu.async_copy(rbs[cur], xs_hbm.at[idxs[cur]], sems[cur])
            if j + 1 < nchunk:
                off = base + (j + 1) * chunk
                nxt = (j + 1) % 2
                pltpu.sync_copy(pos_hbm.at[pl.ds(off, chunk)], idxs[nxt])
                pltpu.sync_copy(x_hbm.at[pl.ds(off, chunk)], rbs[nxt])
            cp.wait()

    return sc_scatter


def _make_sc_gather(n_rows, cap, d, chunk):
    mesh = plsc.VectorSubcoreMesh(core_axis_name="c", subcore_axis_name="s")
    per_w = n_rows // _NW
    nchunk = per_w // chunk

    @functools.partial(
        pl.kernel,
        out_type=jax.ShapeDtypeStruct((n_rows, d), jnp.float32),
        mesh=mesh,
        scratch_types=[
            pltpu.VMEM((chunk,), jnp.int32),
            pltpu.VMEM((chunk,), jnp.int32),
            pltpu.VMEM((chunk, d), jnp.float32),
            pltpu.VMEM((chunk, d), jnp.float32),
            pltpu.SemaphoreType.DMA,
            pltpu.SemaphoreType.DMA,
        ],
    )
    def sc_gather(ys_hbm, pos_hbm, out_hbm, idx0, idx1, rb0, rb1, sem0, sem1):
        wid = lax.axis_index("s") * 2 + lax.axis_index("c")
        base = wid * per_w
        idxs = (idx0, idx1)
        rbs = (rb0, rb1)
        sems = (sem0, sem1)
        pltpu.sync_copy(pos_hbm.at[pl.ds(base, chunk)], idx0)
        cps = [pltpu.async_copy(ys_hbm.at[idx0], rb0, sem0)]
        for j in range(nchunk):
            cur = j % 2
            if j + 1 < nchunk:
                off = base + (j + 1) * chunk
                nxt = (j + 1) % 2
                pltpu.sync_copy(pos_hbm.at[pl.ds(off, chunk)], idxs[nxt])
                cps.append(
                    pltpu.async_copy(ys_hbm.at[idxs[nxt]], rbs[nxt], sems[nxt]))
            cps[j].wait()
            pltpu.sync_copy(rbs[cur], out_hbm.at[pl.ds(base + j * chunk, chunk)])

    return sc_gather


@jax.jit
def kernel(x, Wr, br, We, be, Wo, bo, norm_w):
    B, S, D = x.shape
    E = Wr.shape[0]
    N = B * S
    NT = N // _T
    CAP = N + _NCOMBO * _T
    GT = CAP // _T

    xf = x.reshape(N, D)
    Wr16 = Wr.astype(jnp.bfloat16)
    We16 = We.astype(jnp.bfloat16)
    Wo16 = Wo.astype(jnp.bfloat16)
    brT = br.reshape(E, 1)
    br2 = br.reshape(1, E)
    bo2 = bo.reshape(1, D)
    nw2 = norm_w.reshape(1, D)

    pos_arr, lo_t, hi_t = pl.pallas_call(
        _router_body,
        grid=(NT + 1,),
        in_specs=[
            pl.BlockSpec((_T, D), lambda i: (jnp.minimum(i, NT - 1), 0)),
            pl.BlockSpec((E, D), lambda i: (0, 0)),
            pl.BlockSpec((E, 1), lambda i: (0, 0)),
        ],
        out_specs=[
            pl.BlockSpec((NT, _T), lambda i: (0, 0)),
            pl.BlockSpec((1, GT), lambda i: (0, 0)),
            pl.BlockSpec((1, GT), lambda i: (0, 0)),
        ],
        out_shape=[
            jax.ShapeDtypeStruct((NT, _T), jnp.int32),
            jax.ShapeDtypeStruct((1, GT), jnp.int32),
            jax.ShapeDtypeStruct((1, GT), jnp.int32),
        ],
        scratch_shapes=[
            pltpu.VMEM((NT, _T), jnp.int32),
            pltpu.VMEM((NT, _T), jnp.int32),
        ],
    )(xf, Wr16, brT)

    pos = pos_arr.reshape(N)
    xs = _make_sc_scatter(N, CAP, D, 16)(xf, pos)

    ys = pl.pallas_call(
        _expert_body,
        grid_spec=pltpu.PrefetchScalarGridSpec(
            num_scalar_prefetch=2,
            grid=(GT,),
            in_specs=[
                pl.BlockSpec((_T, D), lambda i, lo, hi: (i, 0)),
                pl.BlockSpec((E, D), lambda i, lo, hi: (0, 0)),
                pl.BlockSpec((1, E), lambda i, lo, hi: (0, 0)),
                pl.BlockSpec((1, D, D), lambda i, lo, hi: (lo[i], 0, 0)),
                pl.BlockSpec((1, D, D), lambda i, lo, hi: (hi[i], 0, 0)),
                pl.BlockSpec((E, D), lambda i, lo, hi: (0, 0)),
                pl.BlockSpec((D, D), lambda i, lo, hi: (0, 0)),
                pl.BlockSpec((1, D), lambda i, lo, hi: (0, 0)),
                pl.BlockSpec((1, D), lambda i, lo, hi: (0, 0)),
            ],
            out_specs=pl.BlockSpec((_T, D), lambda i, lo, hi: (i, 0)),
        ),
        out_shape=jax.ShapeDtypeStruct((CAP, D), jnp.float32),
    )(lo_t.reshape(GT), hi_t.reshape(GT),
      xs, Wr16, br2, We16, We16, be, Wo16, bo2, nw2)

    out = _make_sc_gather(N, CAP, D, 16)(ys, pos)
    return out.reshape(B, S, D)


# speedup vs baseline: 1.0325x; 1.0223x over previous
import functools

import jax
import jax.numpy as jnp
from jax import lax
from jax.experimental import pallas as pl
from jax.experimental.pallas import tpu as pltpu
from jax.experimental.pallas import tpu_sc as plsc

_EPS = 1e-6
_T = 256
_NCOMBO = 6
_NW = 32


def _router_body(x_ref, wr_ref, br_ref, pos_ref, lo_ref, hi_ref, nv_ref,
                 cmb_s, rnk_s):
    i = pl.program_id(0)
    NT = pl.num_programs(0) - 1
    E, D = wr_ref.shape
    T = x_ref.shape[0]
    GT = lo_ref.shape[1]

    @pl.when(i < NT)
    def _route_tile():
        xb = x_ref[:].astype(jnp.bfloat16)
        logt = lax.dot_general(
            wr_ref[:], xb, (((1,), (1,)), ((), ())),
            preferred_element_type=jnp.float32) + br_ref[:]
        iota_e = lax.broadcasted_iota(jnp.int32, (E, T), 0)
        v1 = jnp.max(logt, axis=0, keepdims=True)
        i1 = jnp.min(jnp.where(logt == v1, iota_e, E), axis=0, keepdims=True)
        masked = jnp.where(iota_e == i1, -jnp.inf, logt)
        v2 = jnp.max(masked, axis=0, keepdims=True)
        i2 = jnp.min(jnp.where(masked == v2, iota_e, E), axis=0, keepdims=True)
        lo = jnp.minimum(i1, i2)
        hi = jnp.maximum(i1, i2)
        c = lo * (7 - lo) // 2 + hi - lo - 1
        iota_c = lax.broadcasted_iota(jnp.int32, (_NCOMBO, T), 0)
        oh = (iota_c == c).astype(jnp.int32)
        inc = oh
        for sh in (1, 2, 4, 8, 16, 32, 64, 128):
            if sh < T:
                inc = inc + jnp.concatenate(
                    [jnp.zeros((_NCOMBO, sh), jnp.int32), inc[:, :-sh]], axis=1)
        excl = inc - oh
        rank = jnp.sum(jnp.where(iota_c == c, excl, 0), axis=0, keepdims=True)
        cmb_s[pl.ds(i, 1), :] = c
        rnk_s[pl.ds(i, 1), :] = rank

    @pl.when(i == NT)
    def _finalize():
        cmb = cmb_s[:]
        rnk = rnk_s[:]
        cols = [jnp.sum((cmb == j).astype(jnp.int32), axis=1, keepdims=True)
                for j in range(_NCOMBO)]
        counts = jnp.concatenate(cols, axis=1)
        inc = counts
        sh = 1
        while sh < NT:
            inc = inc + jnp.concatenate(
                [jnp.zeros((sh, _NCOMBO), jnp.int32), inc[:-sh]], axis=0)
            sh *= 2
        excl_tiles = inc - counts
        totals = inc[NT - 1:NT, :]
        ps = ((totals + (T - 1)) // T) * T
        incp = ps
        for sh in (1, 2, 4):
            incp = incp + jnp.concatenate(
                [jnp.zeros((1, sh), jnp.int32), incp[:, :-sh]], axis=1)
        po = incp - ps
        base = excl_tiles + po
        pos = rnk
        for j in range(_NCOMBO):
            pos = pos + jnp.where(cmb == j, base[:, j:j + 1], 0)
        pos_ref[:] = pos

        end_tiles = (po + ps) // T
        t_iota = lax.broadcasted_iota(jnp.int32, (1, GT), 1)
        cot = jnp.zeros((1, GT), jnp.int32)
        for j in range(_NCOMBO):
            cot = cot + (t_iota >= end_tiles[:, j:j + 1]).astype(jnp.int32)
        iota6 = lax.broadcasted_iota(jnp.int32, (1, _NCOMBO), 1)
        lastc = jnp.max(jnp.where(ps > 0, iota6, 0), axis=1, keepdims=True)
        cot = jnp.minimum(cot, lastc)
        nv_ref[:] = jnp.sum(ps, axis=1, keepdims=True) // T
        lo_t = jnp.where(cot < 3, 0, jnp.where(cot < 5, 1, 2))
        blo = (lo_t * (7 - lo_t)) // 2
        hi_t = cot - blo + lo_t + 1
        lo_ref[:] = lo_t
        hi_ref[:] = hi_t


def _expert_body(lo_sref, hi_sref, nv_sref, x_ref, wr_ref, br_ref,
                 welo_ref, wehi_ref, be_ref, wo_ref, bo_ref, nw_ref, o_ref):
    i = pl.program_id(0)
    lo = lo_sref[i]
    hi = hi_sref[i]
    T, D = x_ref.shape
    E = wr_ref.shape[0]

    @pl.when(i < nv_sref[0])
    def _compute():
        _expert_tile(lo, hi, x_ref, wr_ref, br_ref, welo_ref, wehi_ref,
                     be_ref, wo_ref, bo_ref, nw_ref, o_ref)


def _expert_tile(lo, hi, x_ref, wr_ref, br_ref, welo_ref, wehi_ref,
                 be_ref, wo_ref, bo_ref, nw_ref, o_ref):
    T, D = x_ref.shape
    E = wr_ref.shape[0]

    xf = x_ref[:]
    xb = xf.astype(jnp.bfloat16)
    logits = lax.dot_general(
        xb, wr_ref[:], (((1,), (1,)), ((), ())),
        preferred_element_type=jnp.float32) + br_ref[:]
    idx = lax.broadcasted_iota(jnp.int32, (T, E), 1)
    v1 = jnp.max(logits, axis=1, keepdims=True)
    i1 = jnp.min(jnp.where(logits == v1, idx, E), axis=1, keepdims=True)
    masked = jnp.where(idx == i1, -jnp.inf, logits)
    v2 = jnp.max(masked, axis=1, keepdims=True)
    i2 = jnp.min(jnp.where(masked == v2, idx, E), axis=1, keepdims=True)
    s = jnp.exp(v2 - v1)
    w1 = 1.0 / (1.0 + s)
    w2 = s * w1
    gates = jnp.where(idx == i1, w1, 0.0) + jnp.where(idx == i2, w2, 0.0)
    wlo = jnp.sum(jnp.where(idx == lo, gates, 0.0), axis=1, keepdims=True)
    whi = jnp.sum(jnp.where(idx == hi, gates, 0.0), axis=1, keepdims=True)

    hlo = lax.dot_general(
        xb, welo_ref[0], (((1,), (1,)), ((), ())),
        preferred_element_type=jnp.float32) + be_ref[pl.ds(lo, 1), :]
    hhi = lax.dot_general(
        xb, wehi_ref[0], (((1,), (1,)), ((), ())),
        preferred_element_type=jnp.float32) + be_ref[pl.ds(hi, 1), :]
    h = wlo * (hlo * jax.nn.sigmoid(hlo)) + whi * (hhi * jax.nn.sigmoid(hhi))

    ob = lax.dot_general(
        h.astype(jnp.bfloat16), wo_ref[:], (((1,), (1,)), ((), ())),
        preferred_element_type=jnp.float32) + bo_ref[:]
    y = xf + ob
    r = lax.rsqrt(jnp.mean(y * y, axis=1, keepdims=True) + _EPS)
    o_ref[:] = (nw_ref[:] * y) * r


def _make_sc_scatter(n_rows, cap, d, chunk):
    mesh = plsc.VectorSubcoreMesh(core_axis_name="c", subcore_axis_name="s")
    per_w = n_rows // _NW
    nchunk = per_w // chunk

    @functools.partial(
        pl.kernel,
        out_type=jax.ShapeDtypeStruct((cap, d), jnp.float32),
        mesh=mesh,
        scratch_types=[
            pltpu.VMEM((chunk,), jnp.int32),
            pltpu.VMEM((chunk,), jnp.int32),
            pltpu.VMEM((chunk,), jnp.int32),
            pltpu.VMEM((chunk, d), jnp.float32),
            pltpu.VMEM((chunk, d), jnp.float32),
            pltpu.VMEM((chunk, d), jnp.float32),
            pltpu.SemaphoreType.DMA,
            pltpu.SemaphoreType.DMA,
            pltpu.SemaphoreType.DMA,
        ],
    )
    def sc_scatter(x_hbm, pos_hbm, xs_hbm,
                   idx0, idx1, idx2, rb0, rb1, rb2, sem0, sem1, sem2):
        wid = lax.axis_index("s") * 2 + lax.axis_index("c")
        base = wid * per_w
        idxs = (idx0, idx1, idx2)
        rbs = (rb0, rb1, rb2)
        sems = (sem0, sem1, sem2)
        for j in range(min(2, nchunk)):
            off = base + j * chunk
            pltpu.sync_copy(pos_hbm.at[pl.ds(off, chunk)], idxs[j])
            pltpu.sync_copy(x_hbm.at[pl.ds(off, chunk)], rbs[j])
        cps = []
        for j in range(nchunk):
            b = j % 3
            cps.append(pltpu.async_copy(rbs[b], xs_hbm.at[idxs[b]], sems[b]))
            if j > 0:
                cps[j - 1].wait()
            if j + 2 < nchunk:
                off = base + (j + 2) * chunk
                nb = (j + 2) % 3
                pltpu.sync_copy(pos_hbm.at[pl.ds(off, chunk)], idxs[nb])
                pltpu.sync_copy(x_hbm.at[pl.ds(off, chunk)], rbs[nb])
        cps[nchunk - 1].wait()

    return sc_scatter


def _make_sc_gather(n_rows, cap, d, chunk):
    mesh = plsc.VectorSubcoreMesh(core_axis_name="c", subcore_axis_name="s")
    per_w = n_rows // _NW
    nchunk = per_w // chunk

    @functools.partial(
        pl.kernel,
        out_type=jax.ShapeDtypeStruct((n_rows, d), jnp.float32),
        mesh=mesh,
        scratch_types=[
            pltpu.VMEM((per_w,), jnp.int32),
            pltpu.VMEM((chunk, d), jnp.float32),
            pltpu.VMEM((chunk, d), jnp.float32),
            pltpu.VMEM((chunk, d), jnp.float32),
            pltpu.SemaphoreType.DMA,
            pltpu.SemaphoreType.DMA,
            pltpu.SemaphoreType.DMA,
        ],
    )
    def sc_gather(ys_hbm, pos_hbm, out_hbm, idx_all, rb0, rb1, rb2,
                  sem0, sem1, sem2):
        wid = lax.axis_index("s") * 2 + lax.axis_index("c")
        base = wid * per_w
        rbs = (rb0, rb1, rb2)
        sems = (sem0, sem1, sem2)
        pltpu.sync_copy(pos_hbm.at[pl.ds(base, per_w)], idx_all)

        def start(j):
            return pltpu.async_copy(
                ys_hbm.at[idx_all.at[pl.ds(j * chunk, chunk)]],
                rbs[j % 3], sems[j % 3])

        cps = [start(j) for j in range(min(3, nchunk))]
        for j in range(nchunk):
            cps[j].wait()
            pltpu.sync_copy(rbs[j % 3],
                            out_hbm.at[pl.ds(base + j * chunk, chunk)])
            if j + 3 < nchunk:
                cps.append(start(j + 3))

    return sc_gather


@jax.jit
def kernel(x, Wr, br, We, be, Wo, bo, norm_w):
    B, S, D = x.shape
    E = Wr.shape[0]
    N = B * S
    NT = N // _T
    CAP = N + _NCOMBO * _T
    GT = CAP // _T

    xf = x.reshape(N, D)
    Wr16 = Wr.astype(jnp.bfloat16)
    We16 = We.astype(jnp.bfloat16)
    Wo16 = Wo.astype(jnp.bfloat16)
    brT = br.reshape(E, 1)
    br2 = br.reshape(1, E)
    bo2 = bo.reshape(1, D)
    nw2 = norm_w.reshape(1, D)

    pos_arr, lo_t, hi_t, nv_t = pl.pallas_call(
        _router_body,
        grid=(NT + 1,),
        in_specs=[
            pl.BlockSpec((_T, D), lambda i: (jnp.minimum(i, NT - 1), 0)),
            pl.BlockSpec((E, D), lambda i: (0, 0)),
            pl.BlockSpec((E, 1), lambda i: (0, 0)),
        ],
        out_specs=[
            pl.BlockSpec((NT, _T), lambda i: (0, 0)),
            pl.BlockSpec((1, GT), lambda i: (0, 0)),
            pl.BlockSpec((1, GT), lambda i: (0, 0)),
            pl.BlockSpec((1, 1), lambda i: (0, 0)),
        ],
        out_shape=[
            jax.ShapeDtypeStruct((NT, _T), jnp.int32),
            jax.ShapeDtypeStruct((1, GT), jnp.int32),
            jax.ShapeDtypeStruct((1, GT), jnp.int32),
            jax.ShapeDtypeStruct((1, 1), jnp.int32),
        ],
        scratch_shapes=[
            pltpu.VMEM((NT, _T), jnp.int32),
            pltpu.VMEM((NT, _T), jnp.int32),
        ],
    )(xf, Wr16, brT)

    pos = pos_arr.reshape(N)
    xs = _make_sc_scatter(N, CAP, D, 16)(xf, pos)

    ys = pl.pallas_call(
        _expert_body,
        grid_spec=pltpu.PrefetchScalarGridSpec(
            num_scalar_prefetch=3,
            grid=(GT,),
            in_specs=[
                pl.BlockSpec((_T, D), lambda i, lo, hi, nv: (i, 0)),
                pl.BlockSpec((E, D), lambda i, lo, hi, nv: (0, 0)),
                pl.BlockSpec((1, E), lambda i, lo, hi, nv: (0, 0)),
                pl.BlockSpec((1, D, D), lambda i, lo, hi, nv: (lo[i], 0, 0)),
                pl.BlockSpec((1, D, D), lambda i, lo, hi, nv: (hi[i], 0, 0)),
                pl.BlockSpec((E, D), lambda i, lo, hi, nv: (0, 0)),
                pl.BlockSpec((D, D), lambda i, lo, hi, nv: (0, 0)),
                pl.BlockSpec((1, D), lambda i, lo, hi, nv: (0, 0)),
                pl.BlockSpec((1, D), lambda i, lo, hi, nv: (0, 0)),
            ],
            out_specs=pl.BlockSpec((_T, D), lambda i, lo, hi, nv: (i, 0)),
        ),
        out_shape=jax.ShapeDtypeStruct((CAP, D), jnp.float32),
    )(lo_t.reshape(GT), hi_t.reshape(GT), nv_t.reshape(1),
      xs, Wr16, br2, We16, We16, be, Wo16, bo2, nw2)

    out = _make_sc_gather(N, CAP, D, 16)(ys, pos)
    return out.reshape(B, S, D)


# speedup vs baseline: 1.0354x; 1.0028x over previous
import functools

import jax
import jax.numpy as jnp
from jax import lax
from jax.experimental import pallas as pl
from jax.experimental.pallas import tpu as pltpu
from jax.experimental.pallas import tpu_sc as plsc

_EPS = 1e-6
_T = 256
_NCOMBO = 6
_NW = 32


def _router_body(x_ref, wr_ref, br_ref, pos_ref, lo_ref, hi_ref, nv_ref,
                 cmb_s, rnk_s):
    i = pl.program_id(0)
    NT = pl.num_programs(0) - 1
    E, D = wr_ref.shape
    T = x_ref.shape[0]
    GT = lo_ref.shape[1]

    @pl.when(i < NT)
    def _route_tile():
        xb = x_ref[:].astype(jnp.bfloat16)
        logt = lax.dot_general(
            wr_ref[:], xb, (((1,), (1,)), ((), ())),
            preferred_element_type=jnp.float32) + br_ref[:]
        iota_e = lax.broadcasted_iota(jnp.int32, (E, T), 0)
        v1 = jnp.max(logt, axis=0, keepdims=True)
        i1 = jnp.min(jnp.where(logt == v1, iota_e, E), axis=0, keepdims=True)
        masked = jnp.where(iota_e == i1, -jnp.inf, logt)
        v2 = jnp.max(masked, axis=0, keepdims=True)
        i2 = jnp.min(jnp.where(masked == v2, iota_e, E), axis=0, keepdims=True)
        lo = jnp.minimum(i1, i2)
        hi = jnp.maximum(i1, i2)
        c = lo * (7 - lo) // 2 + hi - lo - 1
        iota_c = lax.broadcasted_iota(jnp.int32, (_NCOMBO, T), 0)
        oh = (iota_c == c).astype(jnp.int32)
        inc = oh
        for sh in (1, 2, 4, 8, 16, 32, 64, 128):
            if sh < T:
                inc = inc + jnp.concatenate(
                    [jnp.zeros((_NCOMBO, sh), jnp.int32), inc[:, :-sh]], axis=1)
        excl = inc - oh
        rank = jnp.sum(jnp.where(iota_c == c, excl, 0), axis=0, keepdims=True)
        cmb_s[pl.ds(i, 1), :] = c
        rnk_s[pl.ds(i, 1), :] = rank

    @pl.when(i == NT)
    def _finalize():
        cmb = cmb_s[:]
        rnk = rnk_s[:]
        cols = [jnp.sum((cmb == j).astype(jnp.int32), axis=1, keepdims=True)
                for j in range(_NCOMBO)]
        counts = jnp.concatenate(cols, axis=1)
        inc = counts
        sh = 1
        while sh < NT:
            inc = inc + jnp.concatenate(
                [jnp.zeros((sh, _NCOMBO), jnp.int32), inc[:-sh]], axis=0)
            sh *= 2
        excl_tiles = inc - counts
        totals = inc[NT - 1:NT, :]
        ps = ((totals + (T - 1)) // T) * T
        incp = ps
        for sh in (1, 2, 4):
            incp = incp + jnp.concatenate(
                [jnp.zeros((1, sh), jnp.int32), incp[:, :-sh]], axis=1)
        po = incp - ps
        base = excl_tiles + po
        pos = rnk
        for j in range(_NCOMBO):
            pos = pos + jnp.where(cmb == j, base[:, j:j + 1], 0)
        pos_ref[:] = pos

        end_tiles = (po + ps) // T
        t_iota = lax.broadcasted_iota(jnp.int32, (1, GT), 1)
        cot = jnp.zeros((1, GT), jnp.int32)
        for j in range(_NCOMBO):
            cot = cot + (t_iota >= end_tiles[:, j:j + 1]).astype(jnp.int32)
        iota6 = lax.broadcasted_iota(jnp.int32, (1, _NCOMBO), 1)
        lastc = jnp.max(jnp.where(ps > 0, iota6, 0), axis=1, keepdims=True)
        cot = jnp.minimum(cot, lastc)
        nv_ref[:] = jnp.sum(ps, axis=1, keepdims=True) // T
        lo_t = jnp.where(cot < 3, 0, jnp.where(cot < 5, 1, 2))
        blo = (lo_t * (7 - lo_t)) // 2
        hi_t = cot - blo + lo_t + 1
        lo_ref[:] = lo_t
        hi_ref[:] = hi_t


def _expert_body(lo_sref, hi_sref, nv_sref, x_ref, wr_ref, br_ref,
                 welo_ref, wehi_ref, be_ref, wo_ref, bo_ref, nw_ref, o_ref):
    i = pl.program_id(0)
    lo = lo_sref[i]
    hi = hi_sref[i]
    T, D = x_ref.shape
    E = wr_ref.shape[0]

    @pl.when(i < nv_sref[0])
    def _compute():
        _expert_tile(lo, hi, x_ref, wr_ref, br_ref, welo_ref, wehi_ref,
                     be_ref, wo_ref, bo_ref, nw_ref, o_ref)


def _expert_tile(lo, hi, x_ref, wr_ref, br_ref, welo_ref, wehi_ref,
                 be_ref, wo_ref, bo_ref, nw_ref, o_ref):
    T, D = x_ref.shape
    E = wr_ref.shape[0]

    xf = x_ref[:]
    xb = xf.astype(jnp.bfloat16)
    logits = lax.dot_general(
        xb, wr_ref[:], (((1,), (1,)), ((), ())),
        preferred_element_type=jnp.float32) + br_ref[:]
    idx = lax.broadcasted_iota(jnp.int32, (T, E), 1)
    v1 = jnp.max(logits, axis=1, keepdims=True)
    i1 = jnp.min(jnp.where(logits == v1, idx, E), axis=1, keepdims=True)
    masked = jnp.where(idx == i1, -jnp.inf, logits)
    v2 = jnp.max(masked, axis=1, keepdims=True)
    i2 = jnp.min(jnp.where(masked == v2, idx, E), axis=1, keepdims=True)
    s = jnp.exp(v2 - v1)
    w1 = 1.0 / (1.0 + s)
    w2 = s * w1
    gates = jnp.where(idx == i1, w1, 0.0) + jnp.where(idx == i2, w2, 0.0)
    wlo = jnp.sum(jnp.where(idx == lo, gates, 0.0), axis=1, keepdims=True)
    whi = jnp.sum(jnp.where(idx == hi, gates, 0.0), axis=1, keepdims=True)

    hlo = lax.dot_general(
        xb, welo_ref[0], (((1,), (1,)), ((), ())),
        preferred_element_type=jnp.float32) + be_ref[pl.ds(lo, 1), :]
    hhi = lax.dot_general(
        xb, wehi_ref[0], (((1,), (1,)), ((), ())),
        preferred_element_type=jnp.float32) + be_ref[pl.ds(hi, 1), :]
    h = wlo * (hlo * jax.nn.sigmoid(hlo)) + whi * (hhi * jax.nn.sigmoid(hhi))

    ob = lax.dot_general(
        h.astype(jnp.bfloat16), wo_ref[:], (((1,), (1,)), ((), ())),
        preferred_element_type=jnp.float32) + bo_ref[:]
    y = xf + ob
    r = lax.rsqrt(jnp.mean(y * y, axis=1, keepdims=True) + _EPS)
    o_ref[:] = (nw_ref[:] * y) * r


def _make_sc_scatter(n_rows, cap, d, chunk):
    mesh = plsc.VectorSubcoreMesh(core_axis_name="c", subcore_axis_name="s")
    per_w = n_rows // _NW
    nchunk = per_w // chunk

    @functools.partial(
        pl.kernel,
        out_type=jax.ShapeDtypeStruct((cap, d), jnp.float32),
        mesh=mesh,
        scratch_types=[
            pltpu.VMEM((per_w // chunk, chunk), jnp.int32),
            pltpu.VMEM((chunk, d), jnp.float32),
            pltpu.VMEM((chunk, d), jnp.float32),
            pltpu.VMEM((chunk, d), jnp.float32),
            pltpu.SemaphoreType.DMA,
            pltpu.SemaphoreType.DMA,
            pltpu.SemaphoreType.DMA,
            pltpu.SemaphoreType.DMA,
            pltpu.SemaphoreType.DMA,
            pltpu.SemaphoreType.DMA,
        ],
    )
    def sc_scatter(x_hbm, pos_hbm, xs_hbm,
                   idx2d, rb0, rb1, rb2, lsem0, lsem1, lsem2,
                   ssem0, ssem1, ssem2):
        wid = lax.axis_index("s") * 2 + lax.axis_index("c")
        base = wid * per_w
        rbs = (rb0, rb1, rb2)
        lsems = (lsem0, lsem1, lsem2)
        ssems = (ssem0, ssem1, ssem2)
        icps = [pltpu.async_copy(pos_hbm.at[pl.ds(base + j * chunk, chunk)],
                                 idx2d.at[j], lsems[j % 3])
                for j in range(nchunk)]
        for c in icps:
            c.wait()

        def load(j):
            return pltpu.async_copy(x_hbm.at[pl.ds(base + j * chunk, chunk)],
                                    rbs[j % 3], lsems[j % 3])

        lcp = {j: load(j) for j in range(min(2, nchunk))}
        scp = {}
        swaited = set()
        for j in range(nchunk):
            b = j % 3
            lcp[j].wait()
            scp[j] = pltpu.async_copy(rbs[b], xs_hbm.at[idx2d.at[j]], ssems[b])
            if j + 2 < nchunk:
                if j >= 1:
                    scp[j - 1].wait()
                    swaited.add(j - 1)
                lcp[j + 2] = load(j + 2)
        for j in range(nchunk):
            if j not in swaited:
                scp[j].wait()

    return sc_scatter


def _make_sc_gather(n_rows, cap, d, chunk):
    mesh = plsc.VectorSubcoreMesh(core_axis_name="c", subcore_axis_name="s")
    per_w = n_rows // _NW
    nchunk = per_w // chunk

    @functools.partial(
        pl.kernel,
        out_type=jax.ShapeDtypeStruct((n_rows, d), jnp.float32),
        mesh=mesh,
        scratch_types=[
            pltpu.VMEM((per_w,), jnp.int32),
            pltpu.VMEM((chunk, d), jnp.float32),
            pltpu.VMEM((chunk, d), jnp.float32),
            pltpu.VMEM((chunk, d), jnp.float32),
            pltpu.SemaphoreType.DMA,
            pltpu.SemaphoreType.DMA,
            pltpu.SemaphoreType.DMA,
            pltpu.SemaphoreType.DMA,
            pltpu.SemaphoreType.DMA,
            pltpu.SemaphoreType.DMA,
        ],
    )
    def sc_gather(ys_hbm, pos_hbm, out_hbm, idx_all, rb0, rb1, rb2,
                  gsem0, gsem1, gsem2, wsem0, wsem1, wsem2):
        wid = lax.axis_index("s") * 2 + lax.axis_index("c")
        base = wid * per_w
        rbs = (rb0, rb1, rb2)
        gsems = (gsem0, gsem1, gsem2)
        wsems = (wsem0, wsem1, wsem2)
        pltpu.sync_copy(pos_hbm.at[pl.ds(base, per_w)], idx_all)

        def start(j):
            return pltpu.async_copy(
                ys_hbm.at[idx_all.at[pl.ds(j * chunk, chunk)]],
                rbs[j % 3], gsems[j % 3])

        gcp = {j: start(j) for j in range(min(3, nchunk))}
        wcp = {}
        wwaited = set()
        for j in range(nchunk):
            b = j % 3
            gcp[j].wait()
            wcp[j] = pltpu.async_copy(
                rbs[b], out_hbm.at[pl.ds(base + j * chunk, chunk)], wsems[b])
            if j + 3 < nchunk:
                wcp[j].wait()
                wwaited.add(j)
                gcp[j + 3] = start(j + 3)
        for j in range(nchunk):
            if j not in wwaited:
                wcp[j].wait()

    return sc_gather


@jax.jit
def kernel(x, Wr, br, We, be, Wo, bo, norm_w):
    B, S, D = x.shape
    E = Wr.shape[0]
    N = B * S
    NT = N // _T
    CAP = N + _NCOMBO * _T
    GT = CAP // _T

    xf = x.reshape(N, D)
    Wr16 = Wr.astype(jnp.bfloat16)
    We16 = We.astype(jnp.bfloat16)
    Wo16 = Wo.astype(jnp.bfloat16)
    brT = br.reshape(E, 1)
    br2 = br.reshape(1, E)
    bo2 = bo.reshape(1, D)
    nw2 = norm_w.reshape(1, D)

    pos_arr, lo_t, hi_t, nv_t = pl.pallas_call(
        _router_body,
        grid=(NT + 1,),
        in_specs=[
            pl.BlockSpec((_T, D), lambda i: (jnp.minimum(i, NT - 1), 0)),
            pl.BlockSpec((E, D), lambda i: (0, 0)),
            pl.BlockSpec((E, 1), lambda i: (0, 0)),
        ],
        out_specs=[
            pl.BlockSpec((NT, _T), lambda i: (0, 0)),
            pl.BlockSpec((1, GT), lambda i: (0, 0)),
            pl.BlockSpec((1, GT), lambda i: (0, 0)),
            pl.BlockSpec((1, 1), lambda i: (0, 0)),
        ],
        out_shape=[
            jax.ShapeDtypeStruct((NT, _T), jnp.int32),
            jax.ShapeDtypeStruct((1, GT), jnp.int32),
            jax.ShapeDtypeStruct((1, GT), jnp.int32),
            jax.ShapeDtypeStruct((1, 1), jnp.int32),
        ],
        scratch_shapes=[
            pltpu.VMEM((NT, _T), jnp.int32),
            pltpu.VMEM((NT, _T), jnp.int32),
        ],
    )(xf, Wr16, brT)

    pos = pos_arr.reshape(N)
    xs = _make_sc_scatter(N, CAP, D, 16)(xf, pos)

    ys = pl.pallas_call(
        _expert_body,
        grid_spec=pltpu.PrefetchScalarGridSpec(
            num_scalar_prefetch=3,
            grid=(GT,),
            in_specs=[
                pl.BlockSpec((_T, D), lambda i, lo, hi, nv: (i, 0)),
                pl.BlockSpec((E, D), lambda i, lo, hi, nv: (0, 0)),
                pl.BlockSpec((1, E), lambda i, lo, hi, nv: (0, 0)),
                pl.BlockSpec((1, D, D), lambda i, lo, hi, nv: (lo[i], 0, 0)),
                pl.BlockSpec((1, D, D), lambda i, lo, hi, nv: (hi[i], 0, 0)),
                pl.BlockSpec((E, D), lambda i, lo, hi, nv: (0, 0)),
                pl.BlockSpec((D, D), lambda i, lo, hi, nv: (0, 0)),
                pl.BlockSpec((1, D), lambda i, lo, hi, nv: (0, 0)),
                pl.BlockSpec((1, D), lambda i, lo, hi, nv: (0, 0)),
            ],
            out_specs=pl.BlockSpec((_T, D), lambda i, lo, hi, nv: (i, 0)),
        ),
        out_shape=jax.ShapeDtypeStruct((CAP, D), jnp.float32),
    )(lo_t.reshape(GT), hi_t.reshape(GT), nv_t.reshape(1),
      xs, Wr16, br2, We16, We16, be, Wo16, bo2, nw2)

    out = _make_sc_gather(N, CAP, D, 16)(ys, pos)
    return out.reshape(B, S, D)


# speedup vs baseline: 1.0660x; 1.0295x over previous
import functools

import jax
import jax.numpy as jnp
from jax import lax
from jax.experimental import pallas as pl
from jax.experimental.pallas import tpu as pltpu
from jax.experimental.pallas import tpu_sc as plsc

_EPS = 1e-6
_T = 256
_NCOMBO = 6
_NW = 32


def _router_body(x_ref, wr_ref, br_ref, pos_ref, lo_ref, hi_ref, nv_ref,
                 xpk_ref, cmb_s, rnk_s):
    i = pl.program_id(0)
    NT = pl.num_programs(0) - 1
    E, D = wr_ref.shape
    T = x_ref.shape[0]
    GT = lo_ref.shape[1]

    @pl.when(i < NT)
    def _route_tile():
        xb = x_ref[:].astype(jnp.bfloat16)
        a16 = lax.bitcast_convert_type(xb[:, :D // 2], jnp.uint16)
        b16 = lax.bitcast_convert_type(xb[:, D // 2:], jnp.uint16)
        packed = (a16.astype(jnp.uint32)
                  | (b16.astype(jnp.uint32) << 16))
        xpk_ref[:] = lax.bitcast_convert_type(packed, jnp.int32)
        logt = lax.dot_general(
            wr_ref[:], xb, (((1,), (1,)), ((), ())),
            preferred_element_type=jnp.float32) + br_ref[:]
        iota_e = lax.broadcasted_iota(jnp.int32, (E, T), 0)
        v1 = jnp.max(logt, axis=0, keepdims=True)
        i1 = jnp.min(jnp.where(logt == v1, iota_e, E), axis=0, keepdims=True)
        masked = jnp.where(iota_e == i1, -jnp.inf, logt)
        v2 = jnp.max(masked, axis=0, keepdims=True)
        i2 = jnp.min(jnp.where(masked == v2, iota_e, E), axis=0, keepdims=True)
        lo = jnp.minimum(i1, i2)
        hi = jnp.maximum(i1, i2)
        c = lo * (7 - lo) // 2 + hi - lo - 1
        iota_c = lax.broadcasted_iota(jnp.int32, (_NCOMBO, T), 0)
        oh = (iota_c == c).astype(jnp.int32)
        inc = oh
        for sh in (1, 2, 4, 8, 16, 32, 64, 128):
            if sh < T:
                inc = inc + jnp.concatenate(
                    [jnp.zeros((_NCOMBO, sh), jnp.int32), inc[:, :-sh]], axis=1)
        excl = inc - oh
        rank = jnp.sum(jnp.where(iota_c == c, excl, 0), axis=0, keepdims=True)
        cmb_s[pl.ds(i, 1), :] = c
        rnk_s[pl.ds(i, 1), :] = rank

    @pl.when(i == NT)
    def _finalize():
        cmb = cmb_s[:]
        rnk = rnk_s[:]
        cols = [jnp.sum((cmb == j).astype(jnp.int32), axis=1, keepdims=True)
                for j in range(_NCOMBO)]
        counts = jnp.concatenate(cols, axis=1)
        inc = counts
        sh = 1
        while sh < NT:
            inc = inc + jnp.concatenate(
                [jnp.zeros((sh, _NCOMBO), jnp.int32), inc[:-sh]], axis=0)
            sh *= 2
        excl_tiles = inc - counts
        totals = inc[NT - 1:NT, :]
        ps = ((totals + (T - 1)) // T) * T
        incp = ps
        for sh in (1, 2, 4):
            incp = incp + jnp.concatenate(
                [jnp.zeros((1, sh), jnp.int32), incp[:, :-sh]], axis=1)
        po = incp - ps
        base = excl_tiles + po
        pos = rnk
        for j in range(_NCOMBO):
            pos = pos + jnp.where(cmb == j, base[:, j:j + 1], 0)
        pos_ref[:] = pos

        end_tiles = (po + ps) // T
        t_iota = lax.broadcasted_iota(jnp.int32, (1, GT), 1)
        cot = jnp.zeros((1, GT), jnp.int32)
        for j in range(_NCOMBO):
            cot = cot + (t_iota >= end_tiles[:, j:j + 1]).astype(jnp.int32)
        iota6 = lax.broadcasted_iota(jnp.int32, (1, _NCOMBO), 1)
        lastc = jnp.max(jnp.where(ps > 0, iota6, 0), axis=1, keepdims=True)
        cot = jnp.minimum(cot, lastc)
        nv_ref[:] = jnp.sum(ps, axis=1, keepdims=True) // T
        lo_t = jnp.where(cot < 3, 0, jnp.where(cot < 5, 1, 2))
        blo = (lo_t * (7 - lo_t)) // 2
        hi_t = cot - blo + lo_t + 1
        lo_ref[:] = lo_t
        hi_ref[:] = hi_t


def _expert_body(lo_sref, hi_sref, nv_sref, x_ref, wr_ref, br_ref,
                 welo_ref, wehi_ref, be_ref, wo_ref, bo_ref, nw_ref, o_ref):
    i = pl.program_id(0)
    lo = lo_sref[i]
    hi = hi_sref[i]
    T, D = x_ref.shape
    E = wr_ref.shape[0]

    @pl.when(i < nv_sref[0])
    def _compute():
        _expert_tile(lo, hi, x_ref, wr_ref, br_ref, welo_ref, wehi_ref,
                     be_ref, wo_ref, bo_ref, nw_ref, o_ref)


def _expert_tile(lo, hi, x_ref, wr_ref, br_ref, welo_ref, wehi_ref,
                 be_ref, wo_ref, bo_ref, nw_ref, o_ref):
    T = x_ref.shape[0]
    E, D = wr_ref.shape

    packed = lax.bitcast_convert_type(x_ref[:], jnp.uint32)
    a = lax.bitcast_convert_type(
        (packed & jnp.uint32(0xFFFF)).astype(jnp.uint16), jnp.bfloat16)
    b = lax.bitcast_convert_type(
        (packed >> 16).astype(jnp.uint16), jnp.bfloat16)
    xb = jnp.concatenate([a, b], axis=1)
    xf = xb.astype(jnp.float32)
    logits = lax.dot_general(
        xb, wr_ref[:], (((1,), (1,)), ((), ())),
        preferred_element_type=jnp.float32) + br_ref[:]
    idx = lax.broadcasted_iota(jnp.int32, (T, E), 1)
    v1 = jnp.max(logits, axis=1, keepdims=True)
    i1 = jnp.min(jnp.where(logits == v1, idx, E), axis=1, keepdims=True)
    masked = jnp.where(idx == i1, -jnp.inf, logits)
    v2 = jnp.max(masked, axis=1, keepdims=True)
    i2 = jnp.min(jnp.where(masked == v2, idx, E), axis=1, keepdims=True)
    s = jnp.exp(v2 - v1)
    w1 = 1.0 / (1.0 + s)
    w2 = s * w1
    gates = jnp.where(idx == i1, w1, 0.0) + jnp.where(idx == i2, w2, 0.0)
    wlo = jnp.sum(jnp.where(idx == lo, gates, 0.0), axis=1, keepdims=True)
    whi = jnp.sum(jnp.where(idx == hi, gates, 0.0), axis=1, keepdims=True)

    hlo = lax.dot_general(
        xb, welo_ref[0], (((1,), (1,)), ((), ())),
        preferred_element_type=jnp.float32) + be_ref[pl.ds(lo, 1), :]
    hhi = lax.dot_general(
        xb, wehi_ref[0], (((1,), (1,)), ((), ())),
        preferred_element_type=jnp.float32) + be_ref[pl.ds(hi, 1), :]
    h = wlo * (hlo * jax.nn.sigmoid(hlo)) + whi * (hhi * jax.nn.sigmoid(hhi))

    ob = lax.dot_general(
        h.astype(jnp.bfloat16), wo_ref[:], (((1,), (1,)), ((), ())),
        preferred_element_type=jnp.float32) + bo_ref[:]
    y = xf + ob
    r = lax.rsqrt(jnp.mean(y * y, axis=1, keepdims=True) + _EPS)
    o_ref[:] = (nw_ref[:] * y) * r


def _make_sc_scatter(n_rows, cap, d, chunk):
    mesh = plsc.VectorSubcoreMesh(core_axis_name="c", subcore_axis_name="s")
    per_w = n_rows // _NW
    nchunk = per_w // chunk

    @functools.partial(
        pl.kernel,
        out_type=jax.ShapeDtypeStruct((cap, d), jnp.int32),
        mesh=mesh,
        scratch_types=[
            pltpu.VMEM((per_w // chunk, chunk), jnp.int32),
            pltpu.VMEM((chunk, d), jnp.int32),
            pltpu.VMEM((chunk, d), jnp.int32),
            pltpu.VMEM((chunk, d), jnp.int32),
            pltpu.SemaphoreType.DMA,
            pltpu.SemaphoreType.DMA,
            pltpu.SemaphoreType.DMA,
            pltpu.SemaphoreType.DMA,
            pltpu.SemaphoreType.DMA,
            pltpu.SemaphoreType.DMA,
        ],
    )
    def sc_scatter(x_hbm, pos_hbm, xs_hbm,
                   idx2d, rb0, rb1, rb2, lsem0, lsem1, lsem2,
                   ssem0, ssem1, ssem2):
        wid = lax.axis_index("s") * 2 + lax.axis_index("c")
        base = wid * per_w
        rbs = (rb0, rb1, rb2)
        lsems = (lsem0, lsem1, lsem2)
        ssems = (ssem0, ssem1, ssem2)
        icps = [pltpu.async_copy(pos_hbm.at[pl.ds(base + j * chunk, chunk)],
                                 idx2d.at[j], lsems[j % 3])
                for j in range(nchunk)]
        for c in icps:
            c.wait()

        def load(j):
            return pltpu.async_copy(x_hbm.at[pl.ds(base + j * chunk, chunk)],
                                    rbs[j % 3], lsems[j % 3])

        lcp = {j: load(j) for j in range(min(2, nchunk))}
        scp = {}
        swaited = set()
        for j in range(nchunk):
            b = j % 3
            lcp[j].wait()
            scp[j] = pltpu.async_copy(rbs[b], xs_hbm.at[idx2d.at[j]], ssems[b])
            if j + 2 < nchunk:
                if j >= 1:
                    scp[j - 1].wait()
                    swaited.add(j - 1)
                lcp[j + 2] = load(j + 2)
        for j in range(nchunk):
            if j not in swaited:
                scp[j].wait()

    return sc_scatter


def _make_sc_gather(n_rows, cap, d, chunk):
    mesh = plsc.VectorSubcoreMesh(core_axis_name="c", subcore_axis_name="s")
    per_w = n_rows // _NW
    nchunk = per_w // chunk

    @functools.partial(
        pl.kernel,
        out_type=jax.ShapeDtypeStruct((n_rows, d), jnp.float32),
        mesh=mesh,
        scratch_types=[
            pltpu.VMEM((per_w,), jnp.int32),
            pltpu.VMEM((chunk, d), jnp.float32),
            pltpu.VMEM((chunk, d), jnp.float32),
            pltpu.VMEM((chunk, d), jnp.float32),
            pltpu.SemaphoreType.DMA,
            pltpu.SemaphoreType.DMA,
            pltpu.SemaphoreType.DMA,
            pltpu.SemaphoreType.DMA,
            pltpu.SemaphoreType.DMA,
            pltpu.SemaphoreType.DMA,
        ],
    )
    def sc_gather(ys_hbm, pos_hbm, out_hbm, idx_all, rb0, rb1, rb2,
                  gsem0, gsem1, gsem2, wsem0, wsem1, wsem2):
        wid = lax.axis_index("s") * 2 + lax.axis_index("c")
        base = wid * per_w
        rbs = (rb0, rb1, rb2)
        gsems = (gsem0, gsem1, gsem2)
        wsems = (wsem0, wsem1, wsem2)
        pltpu.sync_copy(pos_hbm.at[pl.ds(base, per_w)], idx_all)

        def start(j):
            return pltpu.async_copy(
                ys_hbm.at[idx_all.at[pl.ds(j * chunk, chunk)]],
                rbs[j % 3], gsems[j % 3])

        gcp = {j: start(j) for j in range(min(3, nchunk))}
        wcp = {}
        wwaited = set()
        for j in range(nchunk):
            b = j % 3
            gcp[j].wait()
            wcp[j] = pltpu.async_copy(
                rbs[b], out_hbm.at[pl.ds(base + j * chunk, chunk)], wsems[b])
            if j + 3 < nchunk:
                wcp[j].wait()
                wwaited.add(j)
                gcp[j + 3] = start(j + 3)
        for j in range(nchunk):
            if j not in wwaited:
                wcp[j].wait()

    return sc_gather


@jax.jit
def kernel(x, Wr, br, We, be, Wo, bo, norm_w):
    B, S, D = x.shape
    E = Wr.shape[0]
    N = B * S
    NT = N // _T
    CAP = N + _NCOMBO * _T
    GT = CAP // _T

    xf = x.reshape(N, D)
    Wr16 = Wr.astype(jnp.bfloat16)
    We16 = We.astype(jnp.bfloat16)
    Wo16 = Wo.astype(jnp.bfloat16)
    brT = br.reshape(E, 1)
    br2 = br.reshape(1, E)
    bo2 = bo.reshape(1, D)
    nw2 = norm_w.reshape(1, D)

    pos_arr, lo_t, hi_t, nv_t, xpk = pl.pallas_call(
        _router_body,
        grid=(NT + 1,),
        in_specs=[
            pl.BlockSpec((_T, D), lambda i: (jnp.minimum(i, NT - 1), 0)),
            pl.BlockSpec((E, D), lambda i: (0, 0)),
            pl.BlockSpec((E, 1), lambda i: (0, 0)),
        ],
        out_specs=[
            pl.BlockSpec((NT, _T), lambda i: (0, 0)),
            pl.BlockSpec((1, GT), lambda i: (0, 0)),
            pl.BlockSpec((1, GT), lambda i: (0, 0)),
            pl.BlockSpec((1, 1), lambda i: (0, 0)),
            pl.BlockSpec((_T, D // 2), lambda i: (jnp.minimum(i, NT - 1), 0)),
        ],
        out_shape=[
            jax.ShapeDtypeStruct((NT, _T), jnp.int32),
            jax.ShapeDtypeStruct((1, GT), jnp.int32),
            jax.ShapeDtypeStruct((1, GT), jnp.int32),
            jax.ShapeDtypeStruct((1, 1), jnp.int32),
            jax.ShapeDtypeStruct((N, D // 2), jnp.int32),
        ],
        scratch_shapes=[
            pltpu.VMEM((NT, _T), jnp.int32),
            pltpu.VMEM((NT, _T), jnp.int32),
        ],
    )(xf, Wr16, brT)

    pos = pos_arr.reshape(N)
    xs = _make_sc_scatter(N, CAP, D // 2, 32)(xpk, pos)

    ys = pl.pallas_call(
        _expert_body,
        grid_spec=pltpu.PrefetchScalarGridSpec(
            num_scalar_prefetch=3,
            grid=(GT,),
            in_specs=[
                pl.BlockSpec((_T, D // 2), lambda i, lo, hi, nv: (i, 0)),
                pl.BlockSpec((E, D), lambda i, lo, hi, nv: (0, 0)),
                pl.BlockSpec((1, E), lambda i, lo, hi, nv: (0, 0)),
                pl.BlockSpec((1, D, D), lambda i, lo, hi, nv: (lo[i], 0, 0)),
                pl.BlockSpec((1, D, D), lambda i, lo, hi, nv: (hi[i], 0, 0)),
                pl.BlockSpec((E, D), lambda i, lo, hi, nv: (0, 0)),
                pl.BlockSpec((D, D), lambda i, lo, hi, nv: (0, 0)),
                pl.BlockSpec((1, D), lambda i, lo, hi, nv: (0, 0)),
                pl.BlockSpec((1, D), lambda i, lo, hi, nv: (0, 0)),
            ],
            out_specs=pl.BlockSpec((_T, D), lambda i, lo, hi, nv: (i, 0)),
        ),
        out_shape=jax.ShapeDtypeStruct((CAP, D), jnp.float32),
    )(lo_t.reshape(GT), hi_t.reshape(GT), nv_t.reshape(1),
      xs, Wr16, br2, We16, We16, be, Wo16, bo2, nw2)

    out = _make_sc_gather(N, CAP, D, 16)(ys, pos)
    return out.reshape(B, S, D)


# speedup vs baseline: 1.0889x; 1.0215x over previous
import functools

import jax
import jax.numpy as jnp
from jax import lax
from jax.experimental import pallas as pl
from jax.experimental.pallas import tpu as pltpu
from jax.experimental.pallas import tpu_sc as plsc

_EPS = 1e-6
_T = 256
_NCOMBO = 6
_NW = 32


def _router_body(x_ref, wr_ref, br_ref, we_ref, pos_ref, lo_ref, hi_ref,
                 nv_ref, xpk_ref, we16_ref, cmb_s, rnk_s):
    i = pl.program_id(0)
    NT = pl.num_programs(0) - 1
    E, D = wr_ref.shape
    T = x_ref.shape[0]
    GT = lo_ref.shape[1]

    we16_ref[:] = we_ref[:].astype(jnp.bfloat16)

    @pl.when(i < NT)
    def _route_tile():
        xb = x_ref[:].astype(jnp.bfloat16)
        a16 = lax.bitcast_convert_type(xb[:, :D // 2], jnp.uint16)
        b16 = lax.bitcast_convert_type(xb[:, D // 2:], jnp.uint16)
        packed = (a16.astype(jnp.uint32)
                  | (b16.astype(jnp.uint32) << 16))
        xpk_ref[:] = lax.bitcast_convert_type(packed, jnp.int32)
        logt = lax.dot_general(
            wr_ref[:], xb, (((1,), (1,)), ((), ())),
            preferred_element_type=jnp.float32) + br_ref[:]
        iota_e = lax.broadcasted_iota(jnp.int32, (E, T), 0)
        v1 = jnp.max(logt, axis=0, keepdims=True)
        i1 = jnp.min(jnp.where(logt == v1, iota_e, E), axis=0, keepdims=True)
        masked = jnp.where(iota_e == i1, -jnp.inf, logt)
        v2 = jnp.max(masked, axis=0, keepdims=True)
        i2 = jnp.min(jnp.where(masked == v2, iota_e, E), axis=0, keepdims=True)
        lo = jnp.minimum(i1, i2)
        hi = jnp.maximum(i1, i2)
        c = lo * (7 - lo) // 2 + hi - lo - 1
        iota_c = lax.broadcasted_iota(jnp.int32, (_NCOMBO, T), 0)
        oh = (iota_c == c).astype(jnp.int32)
        inc = oh
        for sh in (1, 2, 4, 8, 16, 32, 64, 128):
            if sh < T:
                inc = inc + jnp.concatenate(
                    [jnp.zeros((_NCOMBO, sh), jnp.int32), inc[:, :-sh]], axis=1)
        excl = inc - oh
        rank = jnp.sum(jnp.where(iota_c == c, excl, 0), axis=0, keepdims=True)
        cmb_s[pl.ds(i, 1), :] = c
        rnk_s[pl.ds(i, 1), :] = rank

    @pl.when(i == NT)
    def _finalize():
        cmb = cmb_s[:]
        rnk = rnk_s[:]
        cols = [jnp.sum((cmb == j).astype(jnp.int32), axis=1, keepdims=True)
                for j in range(_NCOMBO)]
        counts = jnp.concatenate(cols, axis=1)
        inc = counts
        sh = 1
        while sh < NT:
            inc = inc + jnp.concatenate(
                [jnp.zeros((sh, _NCOMBO), jnp.int32), inc[:-sh]], axis=0)
            sh *= 2
        excl_tiles = inc - counts
        totals = inc[NT - 1:NT, :]
        ps = ((totals + (T - 1)) // T) * T
        incp = ps
        for sh in (1, 2, 4):
            incp = incp + jnp.concatenate(
                [jnp.zeros((1, sh), jnp.int32), incp[:, :-sh]], axis=1)
        po = incp - ps
        base = excl_tiles + po
        pos = rnk
        for j in range(_NCOMBO):
            pos = pos + jnp.where(cmb == j, base[:, j:j + 1], 0)
        pos_ref[:] = pos

        end_tiles = (po + ps) // T
        t_iota = lax.broadcasted_iota(jnp.int32, (1, GT), 1)
        cot = jnp.zeros((1, GT), jnp.int32)
        for j in range(_NCOMBO):
            cot = cot + (t_iota >= end_tiles[:, j:j + 1]).astype(jnp.int32)
        iota6 = lax.broadcasted_iota(jnp.int32, (1, _NCOMBO), 1)
        lastc = jnp.max(jnp.where(ps > 0, iota6, 0), axis=1, keepdims=True)
        cot = jnp.minimum(cot, lastc)
        nv_ref[:] = jnp.sum(ps, axis=1, keepdims=True) // T
        lo_t = jnp.where(cot < 3, 0, jnp.where(cot < 5, 1, 2))
        blo = (lo_t * (7 - lo_t)) // 2
        hi_t = cot - blo + lo_t + 1
        lo_ref[:] = lo_t
        hi_ref[:] = hi_t


def _expert_body(lo_sref, hi_sref, nv_sref, x_ref, wr_ref, br_ref,
                 welo_ref, wehi_ref, be_ref, wo_ref, bo_ref, nw_ref, o_ref):
    i = pl.program_id(0)
    lo = lo_sref[i]
    hi = hi_sref[i]
    T, D = x_ref.shape
    E = wr_ref.shape[0]

    @pl.when(i < nv_sref[0])
    def _compute():
        _expert_tile(lo, hi, x_ref, wr_ref, br_ref, welo_ref, wehi_ref,
                     be_ref, wo_ref, bo_ref, nw_ref, o_ref)


def _expert_tile(lo, hi, x_ref, wr_ref, br_ref, welo_ref, wehi_ref,
                 be_ref, wo_ref, bo_ref, nw_ref, o_ref):
    T = x_ref.shape[0]
    E, D = wr_ref.shape

    packed = lax.bitcast_convert_type(x_ref[:], jnp.uint32)
    a = lax.bitcast_convert_type(
        (packed & jnp.uint32(0xFFFF)).astype(jnp.uint16), jnp.bfloat16)
    b = lax.bitcast_convert_type(
        (packed >> 16).astype(jnp.uint16), jnp.bfloat16)
    xb = jnp.concatenate([a, b], axis=1)
    xf = xb.astype(jnp.float32)
    logits = lax.dot_general(
        xb, wr_ref[:], (((1,), (1,)), ((), ())),
        preferred_element_type=jnp.float32) + br_ref[:]
    idx = lax.broadcasted_iota(jnp.int32, (T, E), 1)
    v1 = jnp.max(logits, axis=1, keepdims=True)
    i1 = jnp.min(jnp.where(logits == v1, idx, E), axis=1, keepdims=True)
    masked = jnp.where(idx == i1, -jnp.inf, logits)
    v2 = jnp.max(masked, axis=1, keepdims=True)
    i2 = jnp.min(jnp.where(masked == v2, idx, E), axis=1, keepdims=True)
    s = jnp.exp(v2 - v1)
    w1 = 1.0 / (1.0 + s)
    w2 = s * w1
    gates = jnp.where(idx == i1, w1, 0.0) + jnp.where(idx == i2, w2, 0.0)
    wlo = jnp.sum(jnp.where(idx == lo, gates, 0.0), axis=1, keepdims=True)
    whi = jnp.sum(jnp.where(idx == hi, gates, 0.0), axis=1, keepdims=True)

    hlo = lax.dot_general(
        xb, welo_ref[0], (((1,), (1,)), ((), ())),
        preferred_element_type=jnp.float32) + be_ref[pl.ds(lo, 1), :]
    hhi = lax.dot_general(
        xb, wehi_ref[0], (((1,), (1,)), ((), ())),
        preferred_element_type=jnp.float32) + be_ref[pl.ds(hi, 1), :]
    h = wlo * (hlo * jax.nn.sigmoid(hlo)) + whi * (hhi * jax.nn.sigmoid(hhi))

    ob = lax.dot_general(
        h.astype(jnp.bfloat16), wo_ref[:], (((1,), (1,)), ((), ())),
        preferred_element_type=jnp.float32) + bo_ref[:]
    y = xf + ob
    r = lax.rsqrt(jnp.mean(y * y, axis=1, keepdims=True) + _EPS)
    o_ref[:] = (nw_ref[:] * y) * r


def _make_sc_scatter(n_rows, cap, d, chunk):
    mesh = plsc.VectorSubcoreMesh(core_axis_name="c", subcore_axis_name="s")
    per_w = n_rows // _NW
    nchunk = per_w // chunk

    @functools.partial(
        pl.kernel,
        out_type=jax.ShapeDtypeStruct((cap, d), jnp.int32),
        mesh=mesh,
        scratch_types=[
            pltpu.VMEM((per_w // chunk, chunk), jnp.int32),
            pltpu.VMEM((chunk, d), jnp.int32),
            pltpu.VMEM((chunk, d), jnp.int32),
            pltpu.VMEM((chunk, d), jnp.int32),
            pltpu.SemaphoreType.DMA,
            pltpu.SemaphoreType.DMA,
            pltpu.SemaphoreType.DMA,
            pltpu.SemaphoreType.DMA,
            pltpu.SemaphoreType.DMA,
            pltpu.SemaphoreType.DMA,
        ],
    )
    def sc_scatter(x_hbm, pos_hbm, xs_hbm,
                   idx2d, rb0, rb1, rb2, lsem0, lsem1, lsem2,
                   ssem0, ssem1, ssem2):
        wid = lax.axis_index("s") * 2 + lax.axis_index("c")
        base = wid * per_w
        rbs = (rb0, rb1, rb2)
        lsems = (lsem0, lsem1, lsem2)
        ssems = (ssem0, ssem1, ssem2)
        icps = [pltpu.async_copy(pos_hbm.at[pl.ds(base + j * chunk, chunk)],
                                 idx2d.at[j], lsems[j % 3])
                for j in range(nchunk)]
        for c in icps:
            c.wait()

        def load(j):
            return pltpu.async_copy(x_hbm.at[pl.ds(base + j * chunk, chunk)],
                                    rbs[j % 3], lsems[j % 3])

        lcp = {j: load(j) for j in range(min(2, nchunk))}
        scp = {}
        swaited = set()
        for j in range(nchunk):
            b = j % 3
            lcp[j].wait()
            scp[j] = pltpu.async_copy(rbs[b], xs_hbm.at[idx2d.at[j]], ssems[b])
            if j + 2 < nchunk:
                if j >= 1:
                    scp[j - 1].wait()
                    swaited.add(j - 1)
                lcp[j + 2] = load(j + 2)
        for j in range(nchunk):
            if j not in swaited:
                scp[j].wait()

    return sc_scatter


def _make_sc_gather(n_rows, cap, d, chunk):
    mesh = plsc.VectorSubcoreMesh(core_axis_name="c", subcore_axis_name="s")
    per_w = n_rows // _NW
    nchunk = per_w // chunk

    @functools.partial(
        pl.kernel,
        out_type=jax.ShapeDtypeStruct((n_rows, d), jnp.float32),
        mesh=mesh,
        scratch_types=[
            pltpu.VMEM((per_w,), jnp.int32),
            pltpu.VMEM((chunk, d), jnp.float32),
            pltpu.VMEM((chunk, d), jnp.float32),
            pltpu.VMEM((chunk, d), jnp.float32),
            pltpu.SemaphoreType.DMA,
            pltpu.SemaphoreType.DMA,
            pltpu.SemaphoreType.DMA,
            pltpu.SemaphoreType.DMA,
            pltpu.SemaphoreType.DMA,
            pltpu.SemaphoreType.DMA,
        ],
    )
    def sc_gather(ys_hbm, pos_hbm, out_hbm, idx_all, rb0, rb1, rb2,
                  gsem0, gsem1, gsem2, wsem0, wsem1, wsem2):
        wid = lax.axis_index("s") * 2 + lax.axis_index("c")
        base = wid * per_w
        rbs = (rb0, rb1, rb2)
        gsems = (gsem0, gsem1, gsem2)
        wsems = (wsem0, wsem1, wsem2)
        pltpu.sync_copy(pos_hbm.at[pl.ds(base, per_w)], idx_all)

        def start(j):
            return pltpu.async_copy(
                ys_hbm.at[idx_all.at[pl.ds(j * chunk, chunk)]],
                rbs[j % 3], gsems[j % 3])

        gcp = {j: start(j) for j in range(min(3, nchunk))}
        wcp = {}
        wwaited = set()
        for j in range(nchunk):
            b = j % 3
            gcp[j].wait()
            wcp[j] = pltpu.async_copy(
                rbs[b], out_hbm.at[pl.ds(base + j * chunk, chunk)], wsems[b])
            if j + 3 < nchunk:
                wcp[j].wait()
                wwaited.add(j)
                gcp[j + 3] = start(j + 3)
        for j in range(nchunk):
            if j not in wwaited:
                wcp[j].wait()

    return sc_gather


@jax.jit
def kernel(x, Wr, br, We, be, Wo, bo, norm_w):
    B, S, D = x.shape
    E = Wr.shape[0]
    N = B * S
    NT = N // _T
    CAP = N + _NCOMBO * _T
    GT = CAP // _T

    xf = x.reshape(N, D)
    Wr16 = Wr.astype(jnp.bfloat16)
    Wo16 = Wo.astype(jnp.bfloat16)
    brT = br.reshape(E, 1)
    br2 = br.reshape(1, E)
    bo2 = bo.reshape(1, D)
    nw2 = norm_w.reshape(1, D)

    pos_arr, lo_t, hi_t, nv_t, xpk, We16 = pl.pallas_call(
        _router_body,
        grid=(NT + 1,),
        in_specs=[
            pl.BlockSpec((_T, D), lambda i: (jnp.minimum(i, NT - 1), 0)),
            pl.BlockSpec((E, D), lambda i: (0, 0)),
            pl.BlockSpec((E, 1), lambda i: (0, 0)),
            pl.BlockSpec((E, D // NT, D),
                         lambda i: (0, jnp.minimum(i, NT - 1), 0)),
        ],
        out_specs=[
            pl.BlockSpec((NT, _T), lambda i: (0, 0)),
            pl.BlockSpec((1, GT), lambda i: (0, 0)),
            pl.BlockSpec((1, GT), lambda i: (0, 0)),
            pl.BlockSpec((1, 1), lambda i: (0, 0)),
            pl.BlockSpec((_T, D // 2), lambda i: (jnp.minimum(i, NT - 1), 0)),
            pl.BlockSpec((E, D // NT, D),
                         lambda i: (0, jnp.minimum(i, NT - 1), 0)),
        ],
        out_shape=[
            jax.ShapeDtypeStruct((NT, _T), jnp.int32),
            jax.ShapeDtypeStruct((1, GT), jnp.int32),
            jax.ShapeDtypeStruct((1, GT), jnp.int32),
            jax.ShapeDtypeStruct((1, 1), jnp.int32),
            jax.ShapeDtypeStruct((N, D // 2), jnp.int32),
            jax.ShapeDtypeStruct((E, D, D), jnp.bfloat16),
        ],
        scratch_shapes=[
            pltpu.VMEM((NT, _T), jnp.int32),
            pltpu.VMEM((NT, _T), jnp.int32),
        ],
    )(xf, Wr16, brT, We)

    pos = pos_arr.reshape(N)
    xs = _make_sc_scatter(N, CAP, D // 2, 32)(xpk, pos)

    ys = pl.pallas_call(
        _expert_body,
        grid_spec=pltpu.PrefetchScalarGridSpec(
            num_scalar_prefetch=3,
            grid=(GT,),
            in_specs=[
                pl.BlockSpec((_T, D // 2), lambda i, lo, hi, nv: (i, 0)),
                pl.BlockSpec((E, D), lambda i, lo, hi, nv: (0, 0)),
                pl.BlockSpec((1, E), lambda i, lo, hi, nv: (0, 0)),
                pl.BlockSpec((1, D, D), lambda i, lo, hi, nv: (lo[i], 0, 0)),
                pl.BlockSpec((1, D, D), lambda i, lo, hi, nv: (hi[i], 0, 0)),
                pl.BlockSpec((E, D), lambda i, lo, hi, nv: (0, 0)),
                pl.BlockSpec((D, D), lambda i, lo, hi, nv: (0, 0)),
                pl.BlockSpec((1, D), lambda i, lo, hi, nv: (0, 0)),
                pl.BlockSpec((1, D), lambda i, lo, hi, nv: (0, 0)),
            ],
            out_specs=pl.BlockSpec((_T, D), lambda i, lo, hi, nv: (i, 0)),
        ),
        out_shape=jax.ShapeDtypeStruct((CAP, D), jnp.float32),
    )(lo_t.reshape(GT), hi_t.reshape(GT), nv_t.reshape(1),
      xs, Wr16, br2, We16, We16, be, Wo16, bo2, nw2)

    out = _make_sc_gather(N, CAP, D, 16)(ys, pos)
    return out.reshape(B, S, D)
